# baseline (speedup 1.0000x reference)
import functools

import jax
import jax.numpy as jnp
from jax import lax
from jax.experimental import pallas as pl
from jax.experimental.pallas import tpu as pltpu

EPS = 1e-06
BETA = 0.25
NVQ_W = 1.0
K_CODES = 256
N_CB = 4


def _softplus(x):
    return jnp.where(x > 20.0, x, jnp.log(1.0 + jnp.exp(jnp.minimum(x, 20.0))))


def _pow(x, e):
    return jnp.exp(e * jnp.log(x))


def _kuma_h(z, a, b):
    xs = jnp.clip(jax.nn.sigmoid(z), EPS, 1.0 - EPS)
    inner = jnp.clip(1.0 - _pow(xs, a), EPS, 1.0)
    y = 1.0 - _pow(inner, b)
    return jnp.clip(y, EPS, 1.0 - EPS)


def _kuma_h_inv(y, inv_a, inv_b):
    y = jnp.clip(y, EPS, 1.0 - EPS)
    inner = 1.0 - _pow(jnp.clip(1.0 - y, EPS, 1.0), inv_b)
    inner = jnp.clip(inner, EPS, 1.0 - EPS)
    xv = jnp.clip(_pow(inner, inv_a), EPS, 1.0 - EPS)
    return jnp.log(xv / (1.0 - xv))


def _fused_body(x_ref, w1, b1, w2, b2, w3, b3, w4, b4, w5, b5, w6, b6,
                cb_ref, cbh_ref, cbm_ref, cbl_ref, ar, br,
                out_ref, idx_ref, rq_ref, nvq_ref,
                *, tile, batch):
    i = pl.program_id(0)

    @pl.when(i == 0)
    def _init():
        rq_ref[0, 0] = 0.0
        nvq_ref[0, 0] = 0.0

    f32 = jnp.float32
    x = x_ref[...]
    h = jnp.maximum(jnp.dot(x, w1[...], preferred_element_type=f32) + b1[...], 0.0)
    h = jnp.maximum(jnp.dot(h, w2[...], preferred_element_type=f32) + b2[...], 0.0)
    z = jnp.dot(h, w3[...], preferred_element_type=f32) + b3[...]

    a = _softplus(ar[...]) + EPS
    b = _softplus(br[...]) + EPS
    zp = _kuma_h(z, a, b)

    res = zp
    q_total = jnp.zeros_like(zp)
    rq_sum = jnp.float32(0.0)
    iota = lax.broadcasted_iota(jnp.int32, (tile, K_CODES), 1)
    for k in range(N_CB):
        cbk = cb_ref[k]
        c2 = jnp.sum(cbk * cbk, axis=1)
        r2 = jnp.sum(res * res, axis=1, keepdims=True)
        s = lax.dot_general(res, cbk, (((1,), (1,)), ((), ())),
                            preferred_element_type=f32)
        d = r2 - 2.0 * s + c2[None, :]
        dmin = jnp.min(d, axis=1, keepdims=True)
        idx = jnp.min(jnp.where(d == dmin, iota, K_CODES), axis=1)
        onehot = (iota == idx[:, None]).astype(jnp.bfloat16)
        bf16 = jnp.bfloat16
        hi = cbk.astype(bf16)
        rm = cbk - hi.astype(f32)
        mid = rm.astype(bf16)
        lo = (rm - mid.astype(f32)).astype(bf16)
        q = (jnp.dot(onehot, hi, preferred_element_type=f32)
             + jnp.dot(onehot, mid, preferred_element_type=f32)
             + jnp.dot(onehot, lo, preferred_element_type=f32))
        idx_ref[k, :] = idx
        q_total = q_total + q
        res = res - q
        rq_sum = rq_sum + jnp.sum(res * res)

    inv_a = 1.0 / a
    inv_b = 1.0 / b
    zq = _kuma_h_inv(q_total, inv_a, inv_b)
    recon = _kuma_h_inv(zp, inv_a, inv_b)
    nvq_sum = jnp.sum((recon - z) ** 2)

    h = jnp.maximum(jnp.dot(zq, w4[...], preferred_element_type=f32) + b4[...], 0.0)
    h = jnp.maximum(jnp.dot(h, w5[...], preferred_element_type=f32) + b5[...], 0.0)
    out_ref[...] = jnp.dot(h, w6[...], preferred_element_type=f32) + b6[...]

    e_dim = zp.shape[1]
    rq_ref[0, 0] += rq_sum * ((1.0 + BETA) / (N_CB * batch * e_dim))
    nvq_ref[0, 0] += nvq_sum * (NVQ_W / (batch * e_dim))


def kernel(x, W1, b1, W2, b2, W3, b3, W4, b4, W5, b5, W6, b6,
           cb0, cb1, cb2, cb3, a_raw, b_raw):
    batch, in_dim = x.shape
    e_dim = W3.shape[1]
    tile = 1024
    grid = batch // tile

    cb = jnp.stack([cb0, cb1, cb2, cb3])
    bf16 = jnp.bfloat16
    cb_hi = cb.astype(bf16)
    r1 = cb - cb_hi.astype(jnp.float32)
    cb_mid = r1.astype(bf16)
    r2_ = r1 - cb_mid.astype(jnp.float32)
    cb_lo = r2_.astype(bf16)
    row = lambda v: v.reshape(1, -1)

    full = lambda arr: pl.BlockSpec(arr.shape, lambda i: (0,) * arr.ndim)
    in_specs = [
        pl.BlockSpec((tile, in_dim), lambda i: (i, 0)),
        full(W1), full(row(b1)), full(W2), full(row(b2)),
        full(W3), full(row(b3)), full(W4), full(row(b4)),
        full(W5), full(row(b5)), full(W6), full(row(b6)),
        full(cb), full(cb_hi), full(cb_mid), full(cb_lo),
        full(row(a_raw)), full(row(b_raw)),
    ]
    out_specs = [
        pl.BlockSpec((tile, in_dim), lambda i: (i, 0)),
        pl.BlockSpec((N_CB, tile), lambda i: (0, i)),
        pl.BlockSpec(memory_space=pltpu.SMEM),
        pl.BlockSpec(memory_space=pltpu.SMEM),
    ]
    out_shapes = [
        jax.ShapeDtypeStruct((batch, in_dim), jnp.float32),
        jax.ShapeDtypeStruct((N_CB, batch), jnp.int32),
        jax.ShapeDtypeStruct((1, 1), jnp.float32),
        jax.ShapeDtypeStruct((1, 1), jnp.float32),
    ]

    out, idx_t, rq, nvq = pl.pallas_call(
        functools.partial(_fused_body, tile=tile, batch=batch),
        grid=(grid,),
        in_specs=in_specs,
        out_specs=out_specs,
        out_shape=out_shapes,
    )(x, W1, row(b1), W2, row(b2), W3, row(b3), W4, row(b4),
      W5, row(b5), W6, row(b6), cb, cb_hi, cb_mid, cb_lo,
      row(a_raw), row(b_raw))

    total_loss = (rq[0, 0] + nvq[0, 0]).astype(jnp.float32)
    indices = idx_t.T
    return (out, total_loss, indices)


# speedup vs baseline: 1.0457x; 1.0457x over previous
import functools

import jax
import jax.numpy as jnp
from jax import lax
from jax.experimental import pallas as pl
from jax.experimental.pallas import tpu as pltpu

EPS = 1e-06
BETA = 0.25
NVQ_W = 1.0
K_CODES = 256
N_CB = 4


def _softplus(x):
    return jnp.where(x > 20.0, x, jnp.log(1.0 + jnp.exp(jnp.minimum(x, 20.0))))


def _pow(x, e):
    return jnp.exp(e * jnp.log(x))


def _kuma_h(z, a, b):
    xs = jnp.clip(jax.nn.sigmoid(z), EPS, 1.0 - EPS)
    inner = jnp.clip(1.0 - _pow(xs, a), EPS, 1.0)
    y = 1.0 - _pow(inner, b)
    return jnp.clip(y, EPS, 1.0 - EPS)


def _kuma_h_inv(y, inv_a, inv_b):
    y = jnp.clip(y, EPS, 1.0 - EPS)
    inner = 1.0 - _pow(jnp.clip(1.0 - y, EPS, 1.0), inv_b)
    inner = jnp.clip(inner, EPS, 1.0 - EPS)
    xv = jnp.clip(_pow(inner, inv_a), EPS, 1.0 - EPS)
    return jnp.log(xv / (1.0 - xv))


def _fused_body(x_ref, w1, b1, w2, b2, w3, b3, w4, b4, w5, b5, w6, b6,
                cb_ref, ar, br,
                out_ref, idx_ref, rq_ref, nvq_ref,
                *, tile, batch):
    i = pl.program_id(0)

    @pl.when(i == 0)
    def _init():
        rq_ref[0, 0] = 0.0
        nvq_ref[0, 0] = 0.0

    f32 = jnp.float32
    x = x_ref[...]
    h = jnp.maximum(jnp.dot(x, w1[...], preferred_element_type=f32) + b1[...], 0.0)
    h = jnp.maximum(jnp.dot(h, w2[...], preferred_element_type=f32) + b2[...], 0.0)
    z = jnp.dot(h, w3[...], preferred_element_type=f32) + b3[...]

    a = _softplus(ar[...]) + EPS
    b = _softplus(br[...]) + EPS
    zp = _kuma_h(z, a, b)

    res = zp
    q_total = jnp.zeros_like(zp)
    rq_sum = jnp.float32(0.0)
    iota = lax.broadcasted_iota(jnp.int32, (tile, K_CODES), 1)
    for k in range(N_CB):
        cbk = cb_ref[k]
        c2 = jnp.sum(cbk * cbk, axis=1)
        r2 = jnp.sum(res * res, axis=1, keepdims=True)
        s = lax.dot_general(res, cbk, (((1,), (1,)), ((), ())),
                            preferred_element_type=f32)
        d = r2 - 2.0 * s + c2[None, :]
        dmin = jnp.min(d, axis=1, keepdims=True)
        idx = jnp.min(jnp.where(d == dmin, iota, K_CODES), axis=1)
        onehot = (iota == idx[:, None]).astype(jnp.bfloat16)
        bf16 = jnp.bfloat16
        hi = cbk.astype(bf16)
        rm = cbk - hi.astype(f32)
        mid = rm.astype(bf16)
        lo = (rm - mid.astype(f32)).astype(bf16)
        q = (jnp.dot(onehot, hi, preferred_element_type=f32)
             + jnp.dot(onehot, mid, preferred_element_type=f32)
             + jnp.dot(onehot, lo, preferred_element_type=f32))
        idx_ref[k, :] = idx
        q_total = q_total + q
        res = res - q
        rq_sum = rq_sum + jnp.sum(res * res)

    inv_a = 1.0 / a
    inv_b = 1.0 / b
    zq = _kuma_h_inv(q_total, inv_a, inv_b)
    recon = _kuma_h_inv(zp, inv_a, inv_b)
    nvq_sum = jnp.sum((recon - z) ** 2)

    h = jnp.maximum(jnp.dot(zq, w4[...], preferred_element_type=f32) + b4[...], 0.0)
    h = jnp.maximum(jnp.dot(h, w5[...], preferred_element_type=f32) + b5[...], 0.0)
    out_ref[...] = jnp.dot(h, w6[...], preferred_element_type=f32) + b6[...]

    e_dim = zp.shape[1]
    rq_ref[0, 0] += rq_sum * ((1.0 + BETA) / (N_CB * batch * e_dim))
    nvq_ref[0, 0] += nvq_sum * (NVQ_W / (batch * e_dim))


def kernel(x, W1, b1, W2, b2, W3, b3, W4, b4, W5, b5, W6, b6,
           cb0, cb1, cb2, cb3, a_raw, b_raw):
    batch, in_dim = x.shape
    e_dim = W3.shape[1]
    tile = 2048
    grid = batch // tile

    cb = jnp.stack([cb0, cb1, cb2, cb3])
    row = lambda v: v.reshape(1, -1)

    full = lambda arr: pl.BlockSpec(arr.shape, lambda i: (0,) * arr.ndim)
    in_specs = [
        pl.BlockSpec((tile, in_dim), lambda i: (i, 0)),
        full(W1), full(row(b1)), full(W2), full(row(b2)),
        full(W3), full(row(b3)), full(W4), full(row(b4)),
        full(W5), full(row(b5)), full(W6), full(row(b6)),
        full(cb), full(row(a_raw)), full(row(b_raw)),
    ]
    out_specs = [
        pl.BlockSpec((tile, in_dim), lambda i: (i, 0)),
        pl.BlockSpec((N_CB, tile), lambda i: (0, i)),
        pl.BlockSpec(memory_space=pltpu.SMEM),
        pl.BlockSpec(memory_space=pltpu.SMEM),
    ]
    out_shapes = [
        jax.ShapeDtypeStruct((batch, in_dim), jnp.float32),
        jax.ShapeDtypeStruct((N_CB, batch), jnp.int32),
        jax.ShapeDtypeStruct((1, 1), jnp.float32),
        jax.ShapeDtypeStruct((1, 1), jnp.float32),
    ]

    out, idx_t, rq, nvq = pl.pallas_call(
        functools.partial(_fused_body, tile=tile, batch=batch),
        grid=(grid,),
        in_specs=in_specs,
        out_specs=out_specs,
        out_shape=out_shapes,
    )(x, W1, row(b1), W2, row(b2), W3, row(b3), W4, row(b4),
      W5, row(b5), W6, row(b6), cb, row(a_raw), row(b_raw))

    total_loss = (rq[0, 0] + nvq[0, 0]).astype(jnp.float32)
    indices = idx_t.T
    return (out, total_loss, indices)
